# baseline (speedup 1.0000x reference)
import functools

import jax
import jax.numpy as jnp
from jax import lax
from jax.experimental import pallas as pl
from jax.experimental.pallas import tpu as pltpu
from jax.experimental.pallas import tpu_sc as plsc

_LANES = 16
_GATHER_WIN = 128


def _proj_body(img_ref, w_ref, out_ref):
    c, h, w = img_ref.shape[1:]
    im = img_ref[0].reshape(c, h * w)
    out_ref[0] = lax.dot_general(
        im, w_ref[...], (((0,), (1,)), ((), ())),
        preferred_element_type=jnp.float32)


def _fuse_body(np_ref, pts_ref, g_ref, wp_ref, wg_ref, br_ref, bg_ref, out_ref):
    b = pl.program_id(0)
    pts2 = pts_ref[0]
    fused = lax.dot_general(
        pts2, wp_ref[...], (((1,), (1,)), ((), ())),
        preferred_element_type=jnp.float32) + g_ref[0] + br_ref[...]
    gate = jax.nn.sigmoid(
        lax.dot_general(
            pts2, wg_ref[...], (((1,), (1,)), ((), ())),
            preferred_element_type=jnp.float32) + bg_ref[...])
    valid = lax.broadcasted_iota(jnp.int32, pts2.shape, 0) < np_ref[b]
    out_ref[0] = jnp.where(valid, fused * gate, 0.0)


def kernel(pts_feats, img_feats, cam_idx, coor_xy, num_points,
           W_reduce, b_reduce, W_gate, b_gate):
    B, P, C = pts_feats.shape
    BN, IC, H, Wd = img_feats.shape
    N = BN // B
    HW = H * Wd
    TOK = B * P
    WIN = _GATHER_WIN

    W_img = W_reduce[:, C:]
    W_pts = W_reduce[:, :C]
    cam_f = cam_idx.reshape(1, TOK)
    x_f = coor_xy[..., 0].reshape(1, TOK)
    y_f = coor_xy[..., 1].reshape(1, TOK)
    bb = ((jnp.arange(TOK, dtype=jnp.int32) // P) * (N * HW)).reshape(1, TOK)

    proj = pl.pallas_call(
        _proj_body,
        grid=(BN,),
        in_specs=[
            pl.BlockSpec((1, IC, H, Wd), lambda n: (n, 0, 0, 0)),
            pl.BlockSpec((C, IC), lambda n: (0, 0)),
        ],
        out_specs=pl.BlockSpec((1, HW, C), lambda n: (n, 0, 0)),
        out_shape=jax.ShapeDtypeStruct((BN, HW, C), jnp.float32),
    )(img_feats, W_img)
    table = proj.reshape(BN * HW, C)

    mesh = plsc.VectorSubcoreMesh(core_axis_name="core",
                                  subcore_axis_name="subcore")

    @functools.partial(
        pl.kernel,
        out_type=jax.ShapeDtypeStruct((TOK, C), jnp.float32),
        mesh=mesh,
        scratch_types=[pltpu.VMEM((WIN,), jnp.int32)],
    )
    def gather_k(table_hbm, cam_hbm, x_hbm, y_hbm, bb_hbm, out_hbm, idx_v):
        def body(cam_v, x_v, y_v, bb_v, o_vmem):
            for k in range(WIN // _LANES):
                s = pl.ds(k * _LANES, _LANES)
                idx_v[s] = (bb_v[0, s] + cam_v[0, s] * HW
                            + y_v[0, s] * Wd + x_v[0, s])
            pltpu.sync_copy(table_hbm.at[idx_v], o_vmem)

        pltpu.emit_pipeline(
            body,
            grid=(TOK // WIN,),
            in_specs=[pl.BlockSpec((1, WIN), lambda i: (0, i))] * 4,
            out_specs=[pl.BlockSpec((WIN, C), lambda i: (i, 0))],
            core_axis_name=("core", "subcore"),
            dimension_semantics=(pltpu.PARALLEL,),
        )(cam_hbm, x_hbm, y_hbm, bb_hbm, out_hbm)

    gathered = gather_k(table, cam_f, x_f, y_f, bb)

    out = pl.pallas_call(
        _fuse_body,
        grid=(B,),
        in_specs=[
            pl.BlockSpec(memory_space=pltpu.SMEM),
            pl.BlockSpec((1, P, C), lambda b: (b, 0, 0)),
            pl.BlockSpec((1, P, C), lambda b: (b, 0, 0)),
            pl.BlockSpec((C, C), lambda b: (0, 0)),
            pl.BlockSpec((C, C), lambda b: (0, 0)),
            pl.BlockSpec((1, C), lambda b: (0, 0)),
            pl.BlockSpec((1, C), lambda b: (0, 0)),
        ],
        out_specs=pl.BlockSpec((1, P, C), lambda b: (b, 0, 0)),
        out_shape=jax.ShapeDtypeStruct((B, P, C), jnp.float32),
    )(num_points, pts_feats, gathered.reshape(B, P, C),
      W_pts, W_gate, b_reduce.reshape(1, C), b_gate.reshape(1, C))
    return out


# speedup vs baseline: 4.5341x; 4.5341x over previous
import functools

import jax
import jax.numpy as jnp
from jax import lax
from jax.experimental import pallas as pl
from jax.experimental.pallas import tpu as pltpu
from jax.experimental.pallas import tpu_sc as plsc

_LANES = 16
_GATHER_WIN = 128


def _pts_body(np_ref, pts_ref, wp_ref, wg_ref, br_ref, bg_ref,
              lin_ref, gate_ref):
    b = pl.program_id(0)
    pts2 = pts_ref[0]
    lin_ref[0] = lax.dot_general(
        pts2, wp_ref[...], (((1,), (1,)), ((), ())),
        preferred_element_type=jnp.float32) + br_ref[...]
    gate = jax.nn.sigmoid(
        lax.dot_general(pts2, wg_ref[...], (((1,), (1,)), ((), ())),
                        preferred_element_type=jnp.float32) + bg_ref[...])
    valid = lax.broadcasted_iota(jnp.int32, pts2.shape, 0) < np_ref[b]
    gate_ref[0] = jnp.where(valid, gate, 0.0)


def _combine_body(lin_ref, g_ref, wi_ref, gate_ref, out_ref):
    img_lin = lax.dot_general(
        g_ref[0], wi_ref[...], (((1,), (1,)), ((), ())),
        preferred_element_type=jnp.float32)
    out_ref[0] = (lin_ref[0] + img_lin) * gate_ref[0]


def kernel(pts_feats, img_feats, cam_idx, coor_xy, num_points,
           W_reduce, b_reduce, W_gate, b_gate):
    B, P, C = pts_feats.shape
    BN, IC, H, Wd = img_feats.shape
    N = BN // B
    HW = H * Wd
    TOK = B * P
    WIN = _GATHER_WIN

    table = jnp.swapaxes(img_feats.reshape(BN, IC, HW), 1, 2).reshape(BN * HW, IC)
    W_pts = W_reduce[:, :C]
    W_img = W_reduce[:, C:]
    cam_f = cam_idx.reshape(1, TOK)
    px = (coor_xy[..., 1] * Wd + coor_xy[..., 0]).reshape(1, TOK) + \
        ((jnp.arange(TOK, dtype=jnp.int32) // P) * (N * HW)).reshape(1, TOK)

    mesh = plsc.VectorSubcoreMesh(core_axis_name="core",
                                  subcore_axis_name="subcore")

    @functools.partial(
        pl.kernel,
        out_type=jax.ShapeDtypeStruct((TOK, IC), jnp.float32),
        mesh=mesh,
        scratch_types=[pltpu.VMEM((WIN,), jnp.int32)],
    )
    def gather_k(table_hbm, cam_hbm, px_hbm, out_hbm, idx_v):
        def body(cam_v, px_v, o_vmem):
            for k in range(WIN // _LANES):
                s = pl.ds(k * _LANES, _LANES)
                idx_v[s] = px_v[0, s] + cam_v[0, s] * HW
            pltpu.sync_copy(table_hbm.at[idx_v], o_vmem)

        pltpu.emit_pipeline(
            body,
            grid=(TOK // WIN,),
            in_specs=[pl.BlockSpec((1, WIN), lambda i: (0, i))] * 2,
            out_specs=[pl.BlockSpec((WIN, IC), lambda i: (i, 0))],
            core_axis_name=("core", "subcore"),
            dimension_semantics=(pltpu.PARALLEL,),
        )(cam_hbm, px_hbm, out_hbm)

    gathered = gather_k(table, cam_f, px)

    pts_lin, gate_m = pl.pallas_call(
        _pts_body,
        grid=(B,),
        in_specs=[
            pl.BlockSpec(memory_space=pltpu.SMEM),
            pl.BlockSpec((1, P, C), lambda b: (b, 0, 0)),
            pl.BlockSpec((C, C), lambda b: (0, 0)),
            pl.BlockSpec((C, C), lambda b: (0, 0)),
            pl.BlockSpec((1, C), lambda b: (0, 0)),
            pl.BlockSpec((1, C), lambda b: (0, 0)),
        ],
        out_specs=[
            pl.BlockSpec((1, P, C), lambda b: (b, 0, 0)),
            pl.BlockSpec((1, P, C), lambda b: (b, 0, 0)),
        ],
        out_shape=[
            jax.ShapeDtypeStruct((B, P, C), jnp.float32),
            jax.ShapeDtypeStruct((B, P, C), jnp.float32),
        ],
    )(num_points, pts_feats, W_pts, W_gate,
      b_reduce.reshape(1, C), b_gate.reshape(1, C))

    out = pl.pallas_call(
        _combine_body,
        grid=(B,),
        in_specs=[
            pl.BlockSpec((1, P, C), lambda b: (b, 0, 0)),
            pl.BlockSpec((1, P, C), lambda b: (b, 0, 0)),
            pl.BlockSpec((C, C), lambda b: (0, 0)),
            pl.BlockSpec((1, P, C), lambda b: (b, 0, 0)),
        ],
        out_specs=pl.BlockSpec((1, P, C), lambda b: (b, 0, 0)),
        out_shape=jax.ShapeDtypeStruct((B, P, C), jnp.float32),
    )(pts_lin, gathered.reshape(B, P, C), W_img, gate_m)
    return out


# speedup vs baseline: 5.3510x; 1.1802x over previous
import functools

import jax
import jax.numpy as jnp
from jax import lax
from jax.experimental import pallas as pl
from jax.experimental.pallas import tpu as pltpu
from jax.experimental.pallas import tpu_sc as plsc

_LANES = 16
_GATHER_WIN = 128
_ROWS = 1024


def _make_fuse_body(P):
    def _fuse_body(np_ref, pts_ref, g_ref, wp_ref, wi_ref, wg_ref, br_ref,
                   bg_ref, out_ref):
        i = pl.program_id(0)
        pts2 = pts_ref[0]
        fused = (
            lax.dot_general(pts2, wp_ref[...], (((1,), (1,)), ((), ())),
                            preferred_element_type=jnp.float32)
            + lax.dot_general(g_ref[0], wi_ref[...], (((1,), (1,)), ((), ())),
                              preferred_element_type=jnp.float32)
            + br_ref[...])
        gate = jax.nn.sigmoid(
            lax.dot_general(pts2, wg_ref[...], (((1,), (1,)), ((), ())),
                            preferred_element_type=jnp.float32) + bg_ref[...])
        row0 = i * pts_ref.shape[1]
        out_ref[0] = jnp.where(
            lax.broadcasted_iota(jnp.int32, pts2.shape, 0) + lax.rem(row0, P)
            < np_ref[row0 // P],
            fused * gate, 0.0)
    return _fuse_body


def kernel(pts_feats, img_feats, cam_idx, coor_xy, num_points,
           W_reduce, b_reduce, W_gate, b_gate):
    B, P, C = pts_feats.shape
    BN, IC, H, Wd = img_feats.shape
    N = BN // B
    HW = H * Wd
    TOK = B * P
    WIN = _GATHER_WIN
    ROWS = _ROWS

    table = jnp.swapaxes(img_feats.reshape(BN, IC, HW), 1, 2).reshape(BN * HW, IC)
    W_pts = W_reduce[:, :C]
    W_img = W_reduce[:, C:]
    cam_f = cam_idx.reshape(1, TOK)
    px = (coor_xy[..., 1] * Wd + coor_xy[..., 0]).reshape(1, TOK) + \
        ((jnp.arange(TOK, dtype=jnp.int32) // P) * (N * HW)).reshape(1, TOK)

    mesh = plsc.VectorSubcoreMesh(core_axis_name="core",
                                  subcore_axis_name="subcore")

    @functools.partial(
        pl.kernel,
        out_type=jax.ShapeDtypeStruct((TOK, IC), jnp.float32),
        mesh=mesh,
        scratch_types=[pltpu.VMEM((WIN,), jnp.int32)],
    )
    def gather_k(table_hbm, cam_hbm, px_hbm, out_hbm, idx_v):
        def body(cam_v, px_v, o_vmem):
            for k in range(WIN // _LANES):
                s = pl.ds(k * _LANES, _LANES)
                idx_v[s] = px_v[0, s] + cam_v[0, s] * HW
            pltpu.sync_copy(table_hbm.at[idx_v], o_vmem)

        pltpu.emit_pipeline(
            body,
            grid=(TOK // WIN,),
            in_specs=[pl.BlockSpec((1, WIN), lambda i: (0, i))] * 2,
            out_specs=[pl.BlockSpec((WIN, IC), lambda i: (i, 0))],
            core_axis_name=("core", "subcore"),
            dimension_semantics=(pltpu.PARALLEL,),
        )(cam_hbm, px_hbm, out_hbm)

    gathered = gather_k(table, cam_f, px)

    grid = TOK // ROWS
    out = pl.pallas_call(
        _make_fuse_body(P),
        grid=(grid,),
        in_specs=[
            pl.BlockSpec(memory_space=pltpu.SMEM),
            pl.BlockSpec((1, ROWS, C), lambda i: (i, 0, 0)),
            pl.BlockSpec((1, ROWS, C), lambda i: (i, 0, 0)),
            pl.BlockSpec((C, C), lambda i: (0, 0)),
            pl.BlockSpec((C, C), lambda i: (0, 0)),
            pl.BlockSpec((C, C), lambda i: (0, 0)),
            pl.BlockSpec((1, C), lambda i: (0, 0)),
            pl.BlockSpec((1, C), lambda i: (0, 0)),
        ],
        out_specs=pl.BlockSpec((1, ROWS, C), lambda i: (i, 0, 0)),
        out_shape=jax.ShapeDtypeStruct((grid, ROWS, C), jnp.float32),
    )(num_points, pts_feats.reshape(grid, ROWS, C),
      gathered.reshape(grid, ROWS, C),
      W_pts, W_img, W_gate, b_reduce.reshape(1, C), b_gate.reshape(1, C))
    return out.reshape(B, P, C)


# speedup vs baseline: 5.8142x; 1.0866x over previous
import functools

import jax
import jax.numpy as jnp
from jax import lax
from jax.experimental import pallas as pl
from jax.experimental.pallas import tpu as pltpu
from jax.experimental.pallas import tpu_sc as plsc

_LANES = 16
_GATHER_WIN = 128
_ROWS = 2048


def _make_fuse_body(P):
    def _fuse_body(np_ref, pts_ref, g_ref, wp_ref, wi_ref, wg_ref, br_ref,
                   bg_ref, out_ref):
        i = pl.program_id(0)
        pts2 = pts_ref[0]
        fused = (
            lax.dot_general(pts2, wp_ref[...], (((1,), (1,)), ((), ())),
                            preferred_element_type=jnp.float32)
            + lax.dot_general(g_ref[0], wi_ref[...], (((1,), (1,)), ((), ())),
                              preferred_element_type=jnp.float32)
            + br_ref[...])
        gate = jax.nn.sigmoid(
            lax.dot_general(pts2, wg_ref[...], (((1,), (1,)), ((), ())),
                            preferred_element_type=jnp.float32) + bg_ref[...])
        row0 = i * pts_ref.shape[1]
        out_ref[0] = jnp.where(
            lax.broadcasted_iota(jnp.int32, pts2.shape, 0) + lax.rem(row0, P)
            < np_ref[row0 // P],
            fused * gate, 0.0)
    return _fuse_body


def kernel(pts_feats, img_feats, cam_idx, coor_xy, num_points,
           W_reduce, b_reduce, W_gate, b_gate):
    B, P, C = pts_feats.shape
    BN, IC, H, Wd = img_feats.shape
    N = BN // B
    HW = H * Wd
    TOK = B * P
    WIN = _GATHER_WIN
    ROWS = _ROWS

    table = jnp.swapaxes(img_feats.reshape(BN, IC, HW), 1, 2).reshape(BN * HW, IC)
    W_pts = W_reduce[:, :C]
    W_img = W_reduce[:, C:]
    cam_f = cam_idx.reshape(1, TOK)
    px = (coor_xy[..., 1] * Wd + coor_xy[..., 0]).reshape(1, TOK) + \
        ((jnp.arange(TOK, dtype=jnp.int32) // P) * (N * HW)).reshape(1, TOK)

    mesh = plsc.VectorSubcoreMesh(core_axis_name="core",
                                  subcore_axis_name="subcore")

    @functools.partial(
        pl.kernel,
        out_type=jax.ShapeDtypeStruct((TOK, IC), jnp.float32),
        mesh=mesh,
        scratch_types=[pltpu.VMEM((WIN,), jnp.int32)],
    )
    def gather_k(table_hbm, cam_hbm, px_hbm, out_hbm, idx_v):
        def body(cam_v, px_v, o_vmem):
            for k in range(WIN // _LANES):
                s = pl.ds(k * _LANES, _LANES)
                idx_v[s] = px_v[0, s] + cam_v[0, s] * HW
            pltpu.sync_copy(table_hbm.at[idx_v], o_vmem)

        pltpu.emit_pipeline(
            body,
            grid=(TOK // WIN,),
            in_specs=[pl.BlockSpec((1, WIN), lambda i: (0, i))] * 2,
            out_specs=[pl.BlockSpec((WIN, IC), lambda i: (i, 0))],
            core_axis_name=("core", "subcore"),
            dimension_semantics=(pltpu.PARALLEL,),
        )(cam_hbm, px_hbm, out_hbm)

    gathered = gather_k(table, cam_f, px)

    grid = TOK // ROWS
    out = pl.pallas_call(
        _make_fuse_body(P),
        grid=(grid,),
        in_specs=[
            pl.BlockSpec(memory_space=pltpu.SMEM),
            pl.BlockSpec((1, ROWS, C), lambda i: (i, 0, 0)),
            pl.BlockSpec((1, ROWS, C), lambda i: (i, 0, 0)),
            pl.BlockSpec((C, C), lambda i: (0, 0)),
            pl.BlockSpec((C, C), lambda i: (0, 0)),
            pl.BlockSpec((C, C), lambda i: (0, 0)),
            pl.BlockSpec((1, C), lambda i: (0, 0)),
            pl.BlockSpec((1, C), lambda i: (0, 0)),
        ],
        out_specs=pl.BlockSpec((1, ROWS, C), lambda i: (i, 0, 0)),
        out_shape=jax.ShapeDtypeStruct((grid, ROWS, C), jnp.float32),
    )(num_points, pts_feats.reshape(grid, ROWS, C),
      gathered.reshape(grid, ROWS, C),
      W_pts, W_img, W_gate, b_reduce.reshape(1, C), b_gate.reshape(1, C))
    return out.reshape(B, P, C)
